# Initial kernel scaffold; baseline (speedup 1.0000x reference)
import functools

import jax
import jax.numpy as jnp
from jax import lax
from jax.experimental import pallas as pl
from jax.experimental.pallas import tpu as pltpu
from jax.experimental.pallas import tpu_sc as plsc

F32 = jnp.float32
I32 = jnp.int32

HH = 6
NN = 50000
EE = 800000
WIN = 256
NWIN = (NN + WIN - 1) // WIN
NPAD = NWIN * WIN
TECS = 32
SUP = 256
CH = 16
EPAD = EE + SUP

GW_WIDE = 784
XU_WIDE = 768
GW_NARR = 112
XU_NARR = 96


def _mm_body(x_ref, w_ref, o_ref):
    o_ref[...] = jnp.dot(x_ref[...], w_ref[...], preferred_element_type=F32)


def _tc_matmul(x, w):
    k, m = x.shape[1], w.shape[1]
    return pl.pallas_call(
        _mm_body,
        grid=(NWIN,),
        in_specs=[pl.BlockSpec((WIN, k), lambda i: (i, 0)),
                  pl.BlockSpec((k, m), lambda i: (0, 0))],
        out_specs=pl.BlockSpec((WIN, m), lambda i: (i, 0)),
        out_shape=jax.ShapeDtypeStruct((NPAD, m), F32),
    )(x, w)


def _fused_body(s_ref, c_ref, b_ref, w_ref, o_ref):
    cnt = jnp.maximum(c_ref[...][:, 0:1], 1.0)
    x = jnp.maximum(s_ref[...] / cnt + b_ref[...], 0.0)
    o_ref[...] = jnp.dot(x, w_ref[...], preferred_element_type=F32)


def _tc_fused(s, cnt, b, w):
    k, m = s.shape[1], w.shape[1]
    return pl.pallas_call(
        _fused_body,
        grid=(NWIN,),
        in_specs=[pl.BlockSpec((WIN, k), lambda i: (i, 0)),
                  pl.BlockSpec((WIN, 16), lambda i: (i, 0)),
                  pl.BlockSpec((1, k), lambda i: (0, 0)),
                  pl.BlockSpec((k, m), lambda i: (0, 0))],
        out_specs=pl.BlockSpec((WIN, m), lambda i: (i, 0)),
        out_shape=jax.ShapeDtypeStruct((NPAD, m), F32),
    )(s, cnt, b, w)


def _final_body(s_ref, c_ref, b_ref, o_ref):
    cnt = jnp.maximum(c_ref[...][:, 0:1], 1.0)
    o_ref[...] = s_ref[...] / cnt + b_ref[...]


def _tc_final(s, cnt, b):
    m = s.shape[1]
    return pl.pallas_call(
        _final_body,
        grid=(NWIN,),
        in_specs=[pl.BlockSpec((WIN, m), lambda i: (i, 0)),
                  pl.BlockSpec((WIN, 16), lambda i: (i, 0)),
                  pl.BlockSpec((1, m), lambda i: (0, 0))],
        out_specs=pl.BlockSpec((WIN, m), lambda i: (i, 0)),
        out_shape=jax.ShapeDtypeStruct((NPAD, m), F32),
    )(s, cnt, b)


@functools.lru_cache(maxsize=None)
def _make_edge_kernel(gw, xu_off, dgrp, hs):
    douts = dgrp * 16
    mesh = plsc.VectorSubcoreMesh(core_axis_name="c", subcore_axis_name="s")

    @functools.partial(
        pl.kernel,
        out_type=(jax.ShapeDtypeStruct((NPAD, douts), F32),
                  jax.ShapeDtypeStruct((NPAD, 16), F32)),
        mesh=mesh,
        scratch_types=[
            pltpu.VMEM((256,), I32),
            pltpu.VMEM((16,), F32),
            pltpu.VMEM((WIN, 16), F32),
            pltpu.VMEM((WIN, douts), F32),
            pltpu.VMEM((WIN, 16), F32),
            pltpu.VMEM((SUP,), I32),
            pltpu.VMEM((SUP,), I32),
            pltpu.VMEM((2, CH, gw), F32),
            pltpu.VMEM((HH, 16), F32),
            pltpu.VMEM((16,), I32),
            pltpu.SemaphoreType.DMA,
            pltpu.SemaphoreType.DMA,
        ],
    )
    def edge_kernel(g_hbm, srcs_hbm, dsts_hbm, offs_hbm, c_hbm,
                    s_hbm, cnt_hbm,
                    offs_v, c_v, xu_v, acc_v, cnt_v, sidv, didv, rows_v,
                    qbuf, dlbuf, sem0, sem1):
        wid = lax.axis_index("s") * 2 + lax.axis_index("c")
        pltpu.sync_copy(offs_hbm, offs_v)
        pltpu.sync_copy(c_hbm, c_v)
        iota16 = lax.iota(I32, 16)
        zeros16 = jnp.zeros((16,), F32)
        onehot = (iota16 == 0).astype(F32)
        c_spl = [lax.broadcast(c_v[h], (16,)) for h in range(HH)]

        def process(j, b, base, n0, e0, e1):
            cb = j * CH
            eb = base + cb
            dloc = didv[pl.ds(cb, CH)] - n0
            dcl = jnp.clip(dloc, 0, WIN - 1)
            xus = [plsc.load_gather(rows_v.at[b],
                                    [iota16, jnp.full((16,), xu_off + h, I32)])
                   for h in range(HH)]
            xud = [plsc.load_gather(xu_v, [dcl, jnp.full((16,), h, I32)])
                   for h in range(HH)]
            t = [xus[h] - xud[h] + c_spl[h] for h in range(HH)]
            mx = jnp.maximum(jnp.maximum(jnp.maximum(t[0], t[1]),
                                         jnp.maximum(t[2], t[3])),
                             jnp.maximum(t[4], t[5]))
            p = [jnp.exp(t[h] - mx) for h in range(HH)]
            r = 1.0 / (p[0] + p[1] + p[2] + p[3] + p[4] + p[5])
            for h in range(HH):
                qbuf[h, :] = p[h] * r
            dlbuf[...] = dloc
            for jj in range(CH):
                valid = jnp.logical_and(eb + jj >= e0, eb + jj < e1)

                def edge_work(jj=jj, b=b):
                    dl = dlbuf[jj]
                    qs = [lax.broadcast(qbuf[h, jj], (16,)) for h in range(HH)]
                    for d in range(dgrp):
                        v = qs[0] * rows_v[b, jj, pl.ds(d * 16, 16)]
                        for h in range(1, HH):
                            v = v + qs[h] * rows_v[b, jj,
                                                   pl.ds(h * hs + d * 16, 16)]
                        plsc.addupdate(acc_v.at[dl, pl.ds(d * 16, 16)], v)
                    plsc.addupdate(cnt_v.at[dl], onehot)

                pl.when(valid)(edge_work)

        def win_body(k, _):
            w = wid + k * TECS
            n0 = w * WIN
            e0 = offs_v[w]
            e1 = offs_v[w + 1]

            def zr(rr, _):
                for d in range(dgrp):
                    acc_v[rr, pl.ds(d * 16, 16)] = zeros16
                cnt_v[rr, pl.ds(0, 16)] = zeros16
                return 0

            lax.fori_loop(0, WIN, zr, 0)
            pltpu.sync_copy(g_hbm.at[pl.ds(n0, WIN), pl.ds(xu_off, 16)], xu_v)
            a0 = (e0 // CH) * CH
            nsub = (e1 - a0 + (SUP - 1)) // SUP

            def sup_body(s, _):
                base = a0 + s * SUP
                pltpu.sync_copy(srcs_hbm.at[pl.ds(base, SUP)], sidv)
                pltpu.sync_copy(dsts_hbm.at[pl.ds(base, SUP)], didv)
                m = jnp.minimum((e1 - base + (CH - 1)) // CH, SUP // CH)

                def pair_body(i, _):
                    j0 = 2 * i
                    j1 = 2 * i + 1
                    j1c = jnp.minimum(j1, m - 1)
                    idx0 = sidv[pl.ds(j0 * CH, CH)]
                    d0 = pltpu.async_copy(g_hbm.at[idx0], rows_v.at[0], sem0)
                    idx1 = sidv[pl.ds(j1c * CH, CH)]
                    d1 = pltpu.async_copy(g_hbm.at[idx1], rows_v.at[1], sem1)
                    d0.wait()
                    process(j0, 0, base, n0, e0, e1)
                    d1.wait()
                    pl.when(j1 < m)(
                        lambda: process(j1c, 1, base, n0, e0, e1))
                    return 0

                lax.fori_loop(0, (m + 1) // 2, pair_body, 0)
                return 0

            lax.fori_loop(0, nsub, sup_body, 0)
            pltpu.sync_copy(acc_v, s_hbm.at[pl.ds(n0, WIN)])
            pltpu.sync_copy(cnt_v, cnt_hbm.at[pl.ds(n0, WIN)])
            return 0

        nw = (NWIN - 1 - wid) // TECS + 1
        lax.fori_loop(0, nw, win_body, 0)

    return edge_kernel


def _edge_stage(g, srcs, dsts, offs, c, narrow):
    if narrow:
        ek = _make_edge_kernel(GW_NARR, XU_NARR, 1, 16)
    else:
        ek = _make_edge_kernel(GW_WIDE, XU_WIDE, 8, 128)
    cpad = jnp.zeros((16,), F32).at[:HH].set(c)
    return ek(g, srcs, dsts, offs, cpad)


def _augment_wide(w, u):
    din = w.shape[0]
    return jnp.concatenate(
        [w, u, jnp.zeros((din, GW_WIDE - XU_WIDE - HH), F32)], axis=1)


def _augment_narrow(w, u):
    din = w.shape[0]
    w3 = w.reshape(din, HH, 3)
    w16 = jnp.pad(w3, ((0, 0), (0, 0), (0, 13))).reshape(din, HH * 16)
    return jnp.concatenate(
        [w16, u, jnp.zeros((din, GW_NARR - XU_NARR - HH), F32)], axis=1)


def kernel(pos, norm, edge_index,
           W1, u1, c1, b1, W2, u2, c2, b2,
           W3, u3, c3, b3, W4, u4, c4, b4):
    src = edge_index[0].astype(I32)
    dst = edge_index[1].astype(I32)

    dst_s, src_s = lax.sort_key_val(dst, src)
    bounds = (jnp.arange(NWIN + 1, dtype=I32) * WIN)
    offs = jnp.searchsorted(dst_s, bounds).astype(I32)
    offs = jnp.concatenate(
        [offs, jnp.full((256 - NWIN - 1,), EE, I32)])
    src_p = jnp.concatenate([src_s, jnp.zeros((EPAD - EE,), I32)])
    dst_p = jnp.concatenate([dst_s, jnp.full((EPAD - EE,), NN, I32)])

    x1 = jnp.concatenate([pos, norm], axis=1)
    x1 = jnp.pad(x1, ((0, NPAD - NN), (0, 2)))
    w1a = jnp.pad(_augment_wide(W1, u1), ((0, 2), (0, 0)))

    g1 = _tc_matmul(x1, w1a)
    s1, cnt = _edge_stage(g1, src_p, dst_p, offs, c1, narrow=False)

    g2 = _tc_fused(s1, cnt, b1.reshape(1, -1), _augment_wide(W2, u2))
    s2, _ = _edge_stage(g2, src_p, dst_p, offs, c2, narrow=False)

    g3 = _tc_fused(s2, cnt, b2.reshape(1, -1), _augment_wide(W3, u3))
    s3, _ = _edge_stage(g3, src_p, dst_p, offs, c3, narrow=False)

    g4 = _tc_fused(s3, cnt, b3.reshape(1, -1), _augment_narrow(W4, u4))
    s4, _ = _edge_stage(g4, src_p, dst_p, offs, c4, narrow=True)

    b4p = jnp.zeros((1, 16), F32).at[0, :3].set(b4)
    out = _tc_final(s4, cnt, b4p)
    return out[:NN, :3]


# speedup vs baseline: 1.2558x; 1.2558x over previous
import functools

import jax
import jax.numpy as jnp
from jax import lax
from jax.experimental import pallas as pl
from jax.experimental.pallas import tpu as pltpu
from jax.experimental.pallas import tpu_sc as plsc

F32 = jnp.float32
I32 = jnp.int32

HH = 6
NN = 50000
EE = 800000
WIN = 256
NWIN = (NN + WIN - 1) // WIN
NPAD = NWIN * WIN
TECS = 32
SUP = 256
CH = 16
EPAD = EE + SUP

GW_WIDE = 896
XU_WIDE = 768
GW_NARR = 128
XU_NARR = 96


def _mm_body(x_ref, w_ref, g_ref, xu_ref, *, xu_off):
    r = jnp.dot(x_ref[...], w_ref[...], preferred_element_type=F32)
    g_ref[...] = r
    xu_ref[...] = r[:, xu_off:xu_off + 16]


def _tc_matmul(x, w, xu_off):
    k, m = x.shape[1], w.shape[1]
    return pl.pallas_call(
        functools.partial(_mm_body, xu_off=xu_off),
        grid=(NWIN,),
        in_specs=[pl.BlockSpec((WIN, k), lambda i: (i, 0)),
                  pl.BlockSpec((k, m), lambda i: (0, 0))],
        out_specs=[pl.BlockSpec((WIN, m), lambda i: (i, 0)),
                   pl.BlockSpec((WIN, 16), lambda i: (i, 0))],
        out_shape=[jax.ShapeDtypeStruct((NPAD, m), F32),
                   jax.ShapeDtypeStruct((NPAD, 16), F32)],
    )(x, w)


def _fused_body(s_ref, c_ref, b_ref, w_ref, g_ref, xu_ref, *, xu_off):
    cnt = jnp.maximum(c_ref[...][:, 0:1], 1.0)
    x = jnp.maximum(s_ref[...] / cnt + b_ref[...], 0.0)
    r = jnp.dot(x, w_ref[...], preferred_element_type=F32)
    g_ref[...] = r
    xu_ref[...] = r[:, xu_off:xu_off + 16]


def _tc_fused(s, cnt, b, w, xu_off):
    k, m = s.shape[1], w.shape[1]
    return pl.pallas_call(
        functools.partial(_fused_body, xu_off=xu_off),
        grid=(NWIN,),
        in_specs=[pl.BlockSpec((WIN, k), lambda i: (i, 0)),
                  pl.BlockSpec((WIN, 16), lambda i: (i, 0)),
                  pl.BlockSpec((1, k), lambda i: (0, 0)),
                  pl.BlockSpec((k, m), lambda i: (0, 0))],
        out_specs=[pl.BlockSpec((WIN, m), lambda i: (i, 0)),
                   pl.BlockSpec((WIN, 16), lambda i: (i, 0))],
        out_shape=[jax.ShapeDtypeStruct((NPAD, m), F32),
                   jax.ShapeDtypeStruct((NPAD, 16), F32)],
    )(s, cnt, b, w)


def _final_body(s_ref, c_ref, b_ref, o_ref):
    cnt = jnp.maximum(c_ref[...][:, 0:1], 1.0)
    o_ref[...] = s_ref[...] / cnt + b_ref[...]


def _tc_final(s, cnt, b):
    m = s.shape[1]
    return pl.pallas_call(
        _final_body,
        grid=(NWIN,),
        in_specs=[pl.BlockSpec((WIN, m), lambda i: (i, 0)),
                  pl.BlockSpec((WIN, 16), lambda i: (i, 0)),
                  pl.BlockSpec((1, m), lambda i: (0, 0))],
        out_specs=pl.BlockSpec((WIN, m), lambda i: (i, 0)),
        out_shape=jax.ShapeDtypeStruct((NPAD, m), F32),
    )(s, cnt, b)


@functools.lru_cache(maxsize=None)
def _make_edge_kernel(gw, xu_off, dgrp, hs):
    douts = dgrp * 16
    mesh = plsc.VectorSubcoreMesh(core_axis_name="c", subcore_axis_name="s")

    @functools.partial(
        pl.kernel,
        out_type=(jax.ShapeDtypeStruct((NPAD, douts), F32),
                  jax.ShapeDtypeStruct((NPAD, 16), F32)),
        mesh=mesh,
        compiler_params=pltpu.CompilerParams(needs_layout_passes=False),
        scratch_types=[
            pltpu.VMEM((256,), I32),
            pltpu.VMEM((16,), F32),
            pltpu.VMEM((WIN, 16), F32),
            pltpu.VMEM((WIN, douts), F32),
            pltpu.VMEM((WIN, 16), F32),
            pltpu.VMEM((SUP,), I32),
            pltpu.VMEM((SUP,), I32),
            pltpu.VMEM((2, CH, gw), F32),
            pltpu.SemaphoreType.DMA,
            pltpu.SemaphoreType.DMA,
        ],
    )
    def edge_kernel(g_hbm, xu_hbm, srcs_hbm, dsts_hbm, offs_hbm, c_hbm,
                    s_hbm, cnt_hbm,
                    offs_v, c_v, xu_v, acc_v, cnt_v, sidv, didv, rows_v,
                    sem0, sem1):
        wid = lax.axis_index("s") * 2 + lax.axis_index("c")
        pltpu.sync_copy(offs_hbm, offs_v)
        pltpu.sync_copy(c_hbm, c_v)
        iota16 = lax.iota(I32, 16)
        zeros16 = jnp.zeros((16,), F32)
        onehot = (iota16 == 0).astype(F32)
        cvec = c_v[pl.ds(0, 16)]
        c_spl = [lax.broadcast(cvec[h], (16,)) for h in range(HH)]

        def process(j, b, base, n0, e0, e1):
            cb = j * CH
            eb = base + cb
            dloc = didv[pl.ds(cb, CH)] - n0
            dcl = jnp.clip(dloc, 0, WIN - 1)
            xus = [plsc.load_gather(rows_v.at[b],
                                    [iota16, jnp.full((16,), xu_off + h, I32)])
                   for h in range(HH)]
            xud = [plsc.load_gather(xu_v, [dcl, jnp.full((16,), h, I32)])
                   for h in range(HH)]
            t = [xus[h] - xud[h] + c_spl[h] for h in range(HH)]
            mx = jnp.maximum(jnp.maximum(jnp.maximum(t[0], t[1]),
                                         jnp.maximum(t[2], t[3])),
                             jnp.maximum(t[4], t[5]))
            p = [jnp.exp(t[h] - mx) for h in range(HH)]
            r = 1.0 / (p[0] + p[1] + p[2] + p[3] + p[4] + p[5])
            q = [p[h] * r for h in range(HH)]
            for jj in range(CH):
                valid = jnp.logical_and(eb + jj >= e0, eb + jj < e1)

                def edge_work(jj=jj, b=b):
                    dl = dloc[jj]
                    qs = [lax.broadcast(q[h][jj], (16,)) for h in range(HH)]
                    for d in range(dgrp):
                        v = qs[0] * rows_v[b, jj, pl.ds(d * 16, 16)]
                        for h in range(1, HH):
                            v = v + qs[h] * rows_v[b, jj,
                                                   pl.ds(h * hs + d * 16, 16)]
                        plsc.addupdate(acc_v.at[dl, pl.ds(d * 16, 16)], v)
                    plsc.addupdate(cnt_v.at[dl], onehot)

                pl.when(valid)(edge_work)

        def win_body(k, _):
            w = wid + k * TECS
            n0 = w * WIN
            ov = offs_v[pl.ds(w, 16)]
            e0 = ov[0]
            e1 = ov[1]

            def zr(rr, _):
                for d in range(dgrp):
                    acc_v[rr, pl.ds(d * 16, 16)] = zeros16
                cnt_v[rr, pl.ds(0, 16)] = zeros16
                return 0

            lax.fori_loop(0, WIN, zr, 0)
            pltpu.sync_copy(xu_hbm.at[pl.ds(n0, WIN)], xu_v)
            a0 = (e0 // CH) * CH
            nsub = (e1 - a0 + (SUP - 1)) // SUP

            def sup_body(s, _):
                base = a0 + s * SUP
                pltpu.sync_copy(srcs_hbm.at[pl.ds(base, SUP)], sidv)
                pltpu.sync_copy(dsts_hbm.at[pl.ds(base, SUP)], didv)
                m = jnp.minimum((e1 - base + (CH - 1)) // CH, SUP // CH)

                def pair_body(i, _):
                    j0 = 2 * i
                    j1 = 2 * i + 1
                    j1c = jnp.minimum(j1, m - 1)
                    idx0 = sidv[pl.ds(j0 * CH, CH)]
                    d0 = pltpu.async_copy(g_hbm.at[idx0], rows_v.at[0], sem0)
                    idx1 = sidv[pl.ds(j1c * CH, CH)]
                    d1 = pltpu.async_copy(g_hbm.at[idx1], rows_v.at[1], sem1)
                    d0.wait()
                    process(j0, 0, base, n0, e0, e1)
                    d1.wait()
                    pl.when(j1 < m)(
                        lambda: process(j1c, 1, base, n0, e0, e1))
                    return 0

                lax.fori_loop(0, (m + 1) // 2, pair_body, 0)
                return 0

            lax.fori_loop(0, nsub, sup_body, 0)
            pltpu.sync_copy(acc_v, s_hbm.at[pl.ds(n0, WIN)])
            pltpu.sync_copy(cnt_v, cnt_hbm.at[pl.ds(n0, WIN)])
            return 0

        nw = (NWIN - 1 - wid) // TECS + 1
        lax.fori_loop(0, nw, win_body, 0)

    return edge_kernel


def _edge_stage(g, xu, srcs, dsts, offs, c, narrow):
    if narrow:
        ek = _make_edge_kernel(GW_NARR, XU_NARR, 1, 16)
    else:
        ek = _make_edge_kernel(GW_WIDE, XU_WIDE, 8, 128)
    cpad = jnp.zeros((16,), F32).at[:HH].set(c)
    return ek(g, xu, srcs, dsts, offs, cpad)


def _augment_wide(w, u):
    din = w.shape[0]
    return jnp.concatenate(
        [w, u, jnp.zeros((din, GW_WIDE - XU_WIDE - HH), F32)], axis=1)


def _augment_narrow(w, u):
    din = w.shape[0]
    w3 = w.reshape(din, HH, 3)
    w16 = jnp.pad(w3, ((0, 0), (0, 0), (0, 13))).reshape(din, HH * 16)
    return jnp.concatenate(
        [w16, u, jnp.zeros((din, GW_NARR - XU_NARR - HH), F32)], axis=1)


def kernel(pos, norm, edge_index,
           W1, u1, c1, b1, W2, u2, c2, b2,
           W3, u3, c3, b3, W4, u4, c4, b4):
    src = edge_index[0].astype(I32)
    dst = edge_index[1].astype(I32)

    dst_s, src_s = lax.sort_key_val(dst, src)
    bounds = (jnp.arange(NWIN + 1, dtype=I32) * WIN)
    offs = jnp.searchsorted(dst_s, bounds).astype(I32)
    offs = jnp.concatenate(
        [offs, jnp.full((256 - NWIN - 1,), EE, I32)])
    src_p = jnp.concatenate([src_s, jnp.zeros((EPAD - EE,), I32)])
    dst_p = jnp.concatenate([dst_s, jnp.full((EPAD - EE,), NN, I32)])

    x1 = jnp.concatenate([pos, norm], axis=1)
    x1 = jnp.pad(x1, ((0, NPAD - NN), (0, 2)))
    w1a = jnp.pad(_augment_wide(W1, u1), ((0, 2), (0, 0)))

    g1, xu1 = _tc_matmul(x1, w1a, XU_WIDE)
    s1, cnt = _edge_stage(g1, xu1, src_p, dst_p, offs, c1, narrow=False)

    g2, xu2 = _tc_fused(s1, cnt, b1.reshape(1, -1), _augment_wide(W2, u2),
                        XU_WIDE)
    s2, _ = _edge_stage(g2, xu2, src_p, dst_p, offs, c2, narrow=False)

    g3, xu3 = _tc_fused(s2, cnt, b2.reshape(1, -1), _augment_wide(W3, u3),
                        XU_WIDE)
    s3, _ = _edge_stage(g3, xu3, src_p, dst_p, offs, c3, narrow=False)

    g4, xu4 = _tc_fused(s3, cnt, b3.reshape(1, -1), _augment_narrow(W4, u4),
                        XU_NARR)
    s4, _ = _edge_stage(g4, xu4, src_p, dst_p, offs, c4, narrow=True)

    b4p = jnp.zeros((1, 16), F32).at[0, :3].set(b4)
    out = _tc_final(s4, cnt, b4p)
    return out[:NN, :3]


# speedup vs baseline: 1.4831x; 1.1810x over previous
import functools

import jax
import jax.numpy as jnp
from jax import lax
from jax.experimental import pallas as pl
from jax.experimental.pallas import tpu as pltpu
from jax.experimental.pallas import tpu_sc as plsc

F32 = jnp.float32
I32 = jnp.int32

HH = 6
NN = 50000
EE = 800000
WIN = 128
NWIN = (NN + WIN - 1) // WIN
NPAD = NWIN * WIN
TCB = 256
NTCB = NPAD // TCB
TECS = 32
SUP = 512
CH = 16
NBUF = 3
EPAD = EE + SUP

GW_WIDE = 896
XU_WIDE = 768
GW_NARR = 128
XU_NARR = 96


def _mm_body(x_ref, w_ref, g_ref, xu_ref, *, xu_off):
    r = jnp.dot(x_ref[...], w_ref[...], preferred_element_type=F32)
    g_ref[...] = r
    xu_ref[...] = r[:, xu_off:xu_off + 16]


def _tc_matmul(x, w, xu_off):
    k, m = x.shape[1], w.shape[1]
    return pl.pallas_call(
        functools.partial(_mm_body, xu_off=xu_off),
        grid=(NTCB,),
        in_specs=[pl.BlockSpec((TCB, k), lambda i: (i, 0)),
                  pl.BlockSpec((k, m), lambda i: (0, 0))],
        out_specs=[pl.BlockSpec((TCB, m), lambda i: (i, 0)),
                   pl.BlockSpec((TCB, 16), lambda i: (i, 0))],
        out_shape=[jax.ShapeDtypeStruct((NPAD, m), F32),
                   jax.ShapeDtypeStruct((NPAD, 16), F32)],
    )(x, w)


def _fused_body(s_ref, c_ref, b_ref, w_ref, g_ref, xu_ref, *, xu_off):
    cnt = jnp.maximum(c_ref[...][:, 0:1], 1.0)
    x = jnp.maximum(s_ref[...] / cnt + b_ref[...], 0.0)
    r = jnp.dot(x, w_ref[...], preferred_element_type=F32)
    g_ref[...] = r
    xu_ref[...] = r[:, xu_off:xu_off + 16]


def _tc_fused(s, cnt, b, w, xu_off):
    k, m = s.shape[1], w.shape[1]
    return pl.pallas_call(
        functools.partial(_fused_body, xu_off=xu_off),
        grid=(NTCB,),
        in_specs=[pl.BlockSpec((TCB, k), lambda i: (i, 0)),
                  pl.BlockSpec((TCB, 16), lambda i: (i, 0)),
                  pl.BlockSpec((1, k), lambda i: (0, 0)),
                  pl.BlockSpec((k, m), lambda i: (0, 0))],
        out_specs=[pl.BlockSpec((TCB, m), lambda i: (i, 0)),
                   pl.BlockSpec((TCB, 16), lambda i: (i, 0))],
        out_shape=[jax.ShapeDtypeStruct((NPAD, m), F32),
                   jax.ShapeDtypeStruct((NPAD, 16), F32)],
    )(s, cnt, b, w)


def _final_body(s_ref, c_ref, b_ref, o_ref):
    cnt = jnp.maximum(c_ref[...][:, 0:1], 1.0)
    o_ref[...] = s_ref[...] / cnt + b_ref[...]


def _tc_final(s, cnt, b):
    m = s.shape[1]
    return pl.pallas_call(
        _final_body,
        grid=(NTCB,),
        in_specs=[pl.BlockSpec((TCB, m), lambda i: (i, 0)),
                  pl.BlockSpec((TCB, 16), lambda i: (i, 0)),
                  pl.BlockSpec((1, m), lambda i: (0, 0))],
        out_specs=pl.BlockSpec((TCB, m), lambda i: (i, 0)),
        out_shape=jax.ShapeDtypeStruct((NPAD, m), F32),
    )(s, cnt, b)


@functools.lru_cache(maxsize=None)
def _make_edge_kernel(gw, xu_off, dgrp, hs):
    douts = dgrp * 16
    mesh = plsc.VectorSubcoreMesh(core_axis_name="c", subcore_axis_name="s")

    @functools.partial(
        pl.kernel,
        out_type=(jax.ShapeDtypeStruct((NPAD, douts), F32),
                  jax.ShapeDtypeStruct((NPAD, 16), F32)),
        mesh=mesh,
        compiler_params=pltpu.CompilerParams(needs_layout_passes=False),
        scratch_types=[
            pltpu.VMEM((512,), I32),
            pltpu.VMEM((16,), F32),
            pltpu.VMEM((WIN, 16), F32),
            pltpu.VMEM((WIN, douts), F32),
            pltpu.VMEM((WIN, 16), F32),
            pltpu.VMEM((SUP,), I32),
            pltpu.VMEM((SUP,), I32),
            pltpu.VMEM((NBUF * CH, gw), F32),
            pltpu.SemaphoreType.DMA,
            pltpu.SemaphoreType.DMA,
            pltpu.SemaphoreType.DMA,
        ],
    )
    def edge_kernel(g_hbm, xu_hbm, srcs_hbm, dsts_hbm, offs_hbm, c_hbm,
                    s_hbm, cnt_hbm,
                    offs_v, c_v, xu_v, acc_v, cnt_v, sidv, didv, rows_v,
                    sem0, sem1, sem2):
        sems = (sem0, sem1, sem2)
        wid = lax.axis_index("s") * 2 + lax.axis_index("c")
        pltpu.sync_copy(offs_hbm, offs_v)
        pltpu.sync_copy(c_hbm, c_v)
        iota16 = lax.iota(I32, 16)
        zeros16 = jnp.zeros((16,), F32)
        onehot = (iota16 == 0).astype(F32)
        cvec = c_v[pl.ds(0, 16)]
        c_spl = [lax.broadcast(cvec[h], (16,)) for h in range(HH)]

        def issue(j, b):
            idx = sidv[pl.ds(j * CH, CH)]
            pltpu.async_copy(g_hbm.at[idx], rows_v.at[pl.ds(b * CH, CH)],
                             sems[b])

        def wait(j, b):
            idx = sidv[pl.ds(j * CH, CH)]
            pltpu.make_async_copy(g_hbm.at[idx],
                                  rows_v.at[pl.ds(b * CH, CH)],
                                  sems[b]).wait()

        def process(j, b, base, n0, e0, e1):
            cb = j * CH
            eb = base + cb
            rb = b * CH
            dloc = didv[pl.ds(cb, CH)] - n0
            dcl = jnp.clip(dloc, 0, WIN - 1)
            rowix = lax.broadcast(rb, (16,)) + iota16
            xus = [plsc.load_gather(rows_v,
                                    [rowix, jnp.full((16,), xu_off + h, I32)])
                   for h in range(HH)]
            xud = [plsc.load_gather(xu_v, [dcl, jnp.full((16,), h, I32)])
                   for h in range(HH)]
            t = [xus[h] - xud[h] + c_spl[h] for h in range(HH)]
            mx = jnp.maximum(jnp.maximum(jnp.maximum(t[0], t[1]),
                                         jnp.maximum(t[2], t[3])),
                             jnp.maximum(t[4], t[5]))
            p = [jnp.exp(t[h] - mx) for h in range(HH)]
            r = 1.0 / (p[0] + p[1] + p[2] + p[3] + p[4] + p[5])
            q = [p[h] * r for h in range(HH)]
            for jj in range(CH):
                valid = jnp.logical_and(eb + jj >= e0, eb + jj < e1)

                def edge_work(jj=jj):
                    dl = dloc[jj]
                    rr = rb + jj
                    qs = [lax.broadcast(q[h][jj], (16,)) for h in range(HH)]
                    for d in range(dgrp):
                        v = qs[0] * rows_v[rr, pl.ds(d * 16, 16)]
                        for h in range(1, HH):
                            v = v + qs[h] * rows_v[rr,
                                                   pl.ds(h * hs + d * 16, 16)]
                        plsc.addupdate(acc_v.at[dl, pl.ds(d * 16, 16)], v)
                    plsc.addupdate(cnt_v.at[dl], onehot)

                pl.when(valid)(edge_work)

        def win_body(k, _):
            w = wid + k * TECS
            n0 = w * WIN
            ov = offs_v[pl.ds(w, 16)]
            e0 = ov[0]
            e1 = ov[1]

            def zr(rr, _):
                for d in range(dgrp):
                    acc_v[rr, pl.ds(d * 16, 16)] = zeros16
                cnt_v[rr, pl.ds(0, 16)] = zeros16
                return 0

            lax.fori_loop(0, WIN, zr, 0)
            pltpu.sync_copy(xu_hbm.at[pl.ds(n0, WIN)], xu_v)
            a0 = (e0 // CH) * CH
            nsub = (e1 - a0 + (SUP - 1)) // SUP

            def sup_body(s, _):
                base = a0 + s * SUP
                pltpu.sync_copy(srcs_hbm.at[pl.ds(base, SUP)], sidv)
                pltpu.sync_copy(dsts_hbm.at[pl.ds(base, SUP)], didv)
                m = jnp.minimum((e1 - base + (CH - 1)) // CH, SUP // CH)
                for b in range(NBUF):
                    issue(jnp.minimum(b, m - 1), b)

                def grp_body(g, _):
                    for b in range(NBUF):
                        j = g * NBUF + b
                        wait(jnp.minimum(j, m - 1), b)
                        pl.when(j < m)(
                            lambda j=j, b=b: process(j, b, base, n0, e0, e1))
                        issue(jnp.minimum(j + NBUF, m - 1), b)
                    return 0

                ngrp = (m + NBUF - 1) // NBUF
                lax.fori_loop(0, ngrp, grp_body, 0)
                for b in range(NBUF):
                    wait(jnp.minimum(ngrp * NBUF + b, m - 1), b)
                return 0

            lax.fori_loop(0, nsub, sup_body, 0)
            pltpu.sync_copy(acc_v, s_hbm.at[pl.ds(n0, WIN)])
            pltpu.sync_copy(cnt_v, cnt_hbm.at[pl.ds(n0, WIN)])
            return 0

        nw = (NWIN - 1 - wid) // TECS + 1
        lax.fori_loop(0, nw, win_body, 0)

    return edge_kernel


def _edge_stage(g, xu, srcs, dsts, offs, c, narrow):
    if narrow:
        ek = _make_edge_kernel(GW_NARR, XU_NARR, 1, 16)
    else:
        ek = _make_edge_kernel(GW_WIDE, XU_WIDE, 8, 128)
    cpad = jnp.zeros((16,), F32).at[:HH].set(c)
    return ek(g, xu, srcs, dsts, offs, cpad)


def _augment_wide(w, u):
    din = w.shape[0]
    return jnp.concatenate(
        [w, u, jnp.zeros((din, GW_WIDE - XU_WIDE - HH), F32)], axis=1)


def _augment_narrow(w, u):
    din = w.shape[0]
    w3 = w.reshape(din, HH, 3)
    w16 = jnp.pad(w3, ((0, 0), (0, 0), (0, 13))).reshape(din, HH * 16)
    return jnp.concatenate(
        [w16, u, jnp.zeros((din, GW_NARR - XU_NARR - HH), F32)], axis=1)


def kernel(pos, norm, edge_index,
           W1, u1, c1, b1, W2, u2, c2, b2,
           W3, u3, c3, b3, W4, u4, c4, b4):
    src = edge_index[0].astype(I32)
    dst = edge_index[1].astype(I32)

    dst_s, src_s = lax.sort_key_val(dst, src)
    bounds = (jnp.arange(NWIN + 1, dtype=I32) * WIN)
    offs = jnp.searchsorted(dst_s, bounds).astype(I32)
    offs = jnp.concatenate(
        [offs, jnp.full((512 - NWIN - 1,), EE, I32)])
    src_p = jnp.concatenate([src_s, jnp.zeros((EPAD - EE,), I32)])
    dst_p = jnp.concatenate([dst_s, jnp.full((EPAD - EE,), NN, I32)])

    x1 = jnp.concatenate([pos, norm], axis=1)
    x1 = jnp.pad(x1, ((0, NPAD - NN), (0, 2)))
    w1a = jnp.pad(_augment_wide(W1, u1), ((0, 2), (0, 0)))

    g1, xu1 = _tc_matmul(x1, w1a, XU_WIDE)
    s1, cnt = _edge_stage(g1, xu1, src_p, dst_p, offs, c1, narrow=False)

    g2, xu2 = _tc_fused(s1, cnt, b1.reshape(1, -1), _augment_wide(W2, u2),
                        XU_WIDE)
    s2, _ = _edge_stage(g2, xu2, src_p, dst_p, offs, c2, narrow=False)

    g3, xu3 = _tc_fused(s2, cnt, b2.reshape(1, -1), _augment_wide(W3, u3),
                        XU_WIDE)
    s3, _ = _edge_stage(g3, xu3, src_p, dst_p, offs, c3, narrow=False)

    g4, xu4 = _tc_fused(s3, cnt, b3.reshape(1, -1), _augment_narrow(W4, u4),
                        XU_NARR)
    s4, _ = _edge_stage(g4, xu4, src_p, dst_p, offs, c4, narrow=True)

    b4p = jnp.zeros((1, 16), F32).at[0, :3].set(b4)
    out = _tc_final(s4, cnt, b4p)
    return out[:NN, :3]


# speedup vs baseline: 1.4962x; 1.0088x over previous
import functools

import jax
import jax.numpy as jnp
from jax import lax
from jax.experimental import pallas as pl
from jax.experimental.pallas import tpu as pltpu
from jax.experimental.pallas import tpu_sc as plsc

F32 = jnp.float32
I32 = jnp.int32

HH = 6
NN = 50000
EE = 800000
WIN = 128
NWIN = (NN + WIN - 1) // WIN
NPAD = NWIN * WIN
TCB = 256
NTCB = NPAD // TCB
TECS = 32
SUP = 1024
CH = 16
NBUF = 4
EPAD = EE + SUP

GW_WIDE = 896
XU_WIDE = 768
GW_NARR = 128
XU_NARR = 96


def _mm_body(x_ref, w_ref, g_ref, xu_ref, *, xu_off):
    r = jnp.dot(x_ref[...], w_ref[...], preferred_element_type=F32)
    g_ref[...] = r
    xu_ref[...] = r[:, xu_off:xu_off + 16]


def _tc_matmul(x, w, xu_off):
    k, m = x.shape[1], w.shape[1]
    return pl.pallas_call(
        functools.partial(_mm_body, xu_off=xu_off),
        grid=(NTCB,),
        in_specs=[pl.BlockSpec((TCB, k), lambda i: (i, 0)),
                  pl.BlockSpec((k, m), lambda i: (0, 0))],
        out_specs=[pl.BlockSpec((TCB, m), lambda i: (i, 0)),
                   pl.BlockSpec((TCB, 16), lambda i: (i, 0))],
        out_shape=[jax.ShapeDtypeStruct((NPAD, m), F32),
                   jax.ShapeDtypeStruct((NPAD, 16), F32)],
    )(x, w)


def _fused_body(s_ref, c_ref, b_ref, w_ref, g_ref, xu_ref, *, xu_off):
    cnt = jnp.maximum(c_ref[...][:, 0:1], 1.0)
    x = jnp.maximum(s_ref[...] / cnt + b_ref[...], 0.0)
    r = jnp.dot(x, w_ref[...], preferred_element_type=F32)
    g_ref[...] = r
    xu_ref[...] = r[:, xu_off:xu_off + 16]


def _tc_fused(s, cnt, b, w, xu_off):
    k, m = s.shape[1], w.shape[1]
    return pl.pallas_call(
        functools.partial(_fused_body, xu_off=xu_off),
        grid=(NTCB,),
        in_specs=[pl.BlockSpec((TCB, k), lambda i: (i, 0)),
                  pl.BlockSpec((TCB, 16), lambda i: (i, 0)),
                  pl.BlockSpec((1, k), lambda i: (0, 0)),
                  pl.BlockSpec((k, m), lambda i: (0, 0))],
        out_specs=[pl.BlockSpec((TCB, m), lambda i: (i, 0)),
                   pl.BlockSpec((TCB, 16), lambda i: (i, 0))],
        out_shape=[jax.ShapeDtypeStruct((NPAD, m), F32),
                   jax.ShapeDtypeStruct((NPAD, 16), F32)],
    )(s, cnt, b, w)


def _final_body(s_ref, c_ref, b_ref, o_ref):
    cnt = jnp.maximum(c_ref[...][:, 0:1], 1.0)
    o_ref[...] = s_ref[...] / cnt + b_ref[...]


def _tc_final(s, cnt, b):
    m = s.shape[1]
    return pl.pallas_call(
        _final_body,
        grid=(NTCB,),
        in_specs=[pl.BlockSpec((TCB, m), lambda i: (i, 0)),
                  pl.BlockSpec((TCB, 16), lambda i: (i, 0)),
                  pl.BlockSpec((1, m), lambda i: (0, 0))],
        out_specs=pl.BlockSpec((TCB, m), lambda i: (i, 0)),
        out_shape=jax.ShapeDtypeStruct((NPAD, m), F32),
    )(s, cnt, b)


@functools.lru_cache(maxsize=None)
def _make_edge_kernel(gw, xu_off, dgrp, hs):
    douts = dgrp * 16
    mesh = plsc.VectorSubcoreMesh(core_axis_name="c", subcore_axis_name="s")

    @functools.partial(
        pl.kernel,
        out_type=(jax.ShapeDtypeStruct((NPAD, douts), F32),
                  jax.ShapeDtypeStruct((NPAD, 16), F32)),
        mesh=mesh,
        compiler_params=pltpu.CompilerParams(needs_layout_passes=False),
        scratch_types=[
            pltpu.VMEM((512,), I32),
            pltpu.VMEM((16,), F32),
            pltpu.VMEM((WIN, 16), F32),
            pltpu.VMEM((WIN, douts), F32),
            pltpu.VMEM((WIN, 16), F32),
            pltpu.VMEM((SUP,), I32),
            pltpu.VMEM((SUP,), I32),
            pltpu.VMEM((NBUF * CH, gw), F32),
            pltpu.SemaphoreType.DMA,
            pltpu.SemaphoreType.DMA,
            pltpu.SemaphoreType.DMA,
            pltpu.SemaphoreType.DMA,
        ],
    )
    def edge_kernel(g_hbm, xu_hbm, srcs_hbm, dsts_hbm, offs_hbm, c_hbm,
                    s_hbm, cnt_hbm,
                    offs_v, c_v, xu_v, acc_v, cnt_v, sidv, didv, rows_v,
                    sem0, sem1, sem2, sem3):
        sems = (sem0, sem1, sem2, sem3)
        wid = lax.axis_index("s") * 2 + lax.axis_index("c")
        pltpu.sync_copy(offs_hbm, offs_v)
        pltpu.sync_copy(c_hbm, c_v)
        iota16 = lax.iota(I32, 16)
        zeros16 = jnp.zeros((16,), F32)
        onehot = (iota16 == 0).astype(F32)
        cvec = c_v[pl.ds(0, 16)]
        c_spl = [lax.broadcast(cvec[h], (16,)) for h in range(HH)]

        def issue(j, b):
            idx = sidv[pl.ds(j * CH, CH)]
            pltpu.async_copy(g_hbm.at[idx], rows_v.at[pl.ds(b * CH, CH)],
                             sems[b])

        def wait(j, b):
            idx = sidv[pl.ds(j * CH, CH)]
            pltpu.make_async_copy(g_hbm.at[idx],
                                  rows_v.at[pl.ds(b * CH, CH)],
                                  sems[b]).wait()

        def process(j, b, base, n0, e0, e1):
            cb = j * CH
            eb = base + cb
            rb = b * CH
            dloc = didv[pl.ds(cb, CH)] - n0
            dcl = jnp.clip(dloc, 0, WIN - 1)
            rowix = lax.broadcast(rb, (16,)) + iota16
            xus = [plsc.load_gather(rows_v,
                                    [rowix, jnp.full((16,), xu_off + h, I32)])
                   for h in range(HH)]
            xud = [plsc.load_gather(xu_v, [dcl, jnp.full((16,), h, I32)])
                   for h in range(HH)]
            t = [xus[h] - xud[h] + c_spl[h] for h in range(HH)]
            mx = jnp.maximum(jnp.maximum(jnp.maximum(t[0], t[1]),
                                         jnp.maximum(t[2], t[3])),
                             jnp.maximum(t[4], t[5]))
            p = [jnp.exp(t[h] - mx) for h in range(HH)]
            r = 1.0 / (p[0] + p[1] + p[2] + p[3] + p[4] + p[5])
            q = [p[h] * r for h in range(HH)]
            for jj in range(CH):
                valid = jnp.logical_and(eb + jj >= e0, eb + jj < e1)

                def edge_work(jj=jj):
                    dl = dloc[jj]
                    rr = rb + jj
                    qs = [lax.broadcast(q[h][jj], (16,)) for h in range(HH)]
                    for d in range(dgrp):
                        v = qs[0] * rows_v[rr, pl.ds(d * 16, 16)]
                        for h in range(1, HH):
                            v = v + qs[h] * rows_v[rr,
                                                   pl.ds(h * hs + d * 16, 16)]
                        plsc.addupdate(acc_v.at[dl, pl.ds(d * 16, 16)], v)
                    plsc.addupdate(cnt_v.at[dl], onehot)

                pl.when(valid)(edge_work)

        def win_body(k, _):
            w = wid + k * TECS
            n0 = w * WIN
            ov = offs_v[pl.ds(w, 16)]
            e0 = ov[0]
            e1 = ov[1]

            def zr(rr, _):
                for d in range(dgrp):
                    acc_v[rr, pl.ds(d * 16, 16)] = zeros16
                cnt_v[rr, pl.ds(0, 16)] = zeros16
                return 0

            lax.fori_loop(0, WIN, zr, 0)
            pltpu.sync_copy(xu_hbm.at[pl.ds(n0, WIN)], xu_v)
            a0 = (e0 // CH) * CH
            nsub = (e1 - a0 + (SUP - 1)) // SUP

            def sup_body(s, _):
                base = a0 + s * SUP
                pltpu.sync_copy(srcs_hbm.at[pl.ds(base, SUP)], sidv)
                pltpu.sync_copy(dsts_hbm.at[pl.ds(base, SUP)], didv)
                m = jnp.minimum((e1 - base + (CH - 1)) // CH, SUP // CH)
                for b in range(NBUF):
                    issue(jnp.minimum(b, m - 1), b)

                def grp_body(g, _):
                    for b in range(NBUF):
                        j = g * NBUF + b
                        wait(jnp.minimum(j, m - 1), b)
                        pl.when(j < m)(
                            lambda j=j, b=b: process(j, b, base, n0, e0, e1))
                        issue(jnp.minimum(j + NBUF, m - 1), b)
                    return 0

                ngrp = (m + NBUF - 1) // NBUF
                lax.fori_loop(0, ngrp, grp_body, 0)
                for b in range(NBUF):
                    wait(jnp.minimum(ngrp * NBUF + b, m - 1), b)
                return 0

            lax.fori_loop(0, nsub, sup_body, 0)
            pltpu.sync_copy(acc_v, s_hbm.at[pl.ds(n0, WIN)])
            pltpu.sync_copy(cnt_v, cnt_hbm.at[pl.ds(n0, WIN)])
            return 0

        nw = (NWIN - 1 - wid) // TECS + 1
        lax.fori_loop(0, nw, win_body, 0)

    return edge_kernel


def _edge_stage(g, xu, srcs, dsts, offs, c, narrow):
    if narrow:
        ek = _make_edge_kernel(GW_NARR, XU_NARR, 1, 16)
    else:
        ek = _make_edge_kernel(GW_WIDE, XU_WIDE, 8, 128)
    cpad = jnp.zeros((16,), F32).at[:HH].set(c)
    return ek(g, xu, srcs, dsts, offs, cpad)


def _augment_wide(w, u):
    din = w.shape[0]
    return jnp.concatenate(
        [w, u, jnp.zeros((din, GW_WIDE - XU_WIDE - HH), F32)], axis=1)


def _augment_narrow(w, u):
    din = w.shape[0]
    w3 = w.reshape(din, HH, 3)
    w16 = jnp.pad(w3, ((0, 0), (0, 0), (0, 13))).reshape(din, HH * 16)
    return jnp.concatenate(
        [w16, u, jnp.zeros((din, GW_NARR - XU_NARR - HH), F32)], axis=1)


def kernel(pos, norm, edge_index,
           W1, u1, c1, b1, W2, u2, c2, b2,
           W3, u3, c3, b3, W4, u4, c4, b4):
    src = edge_index[0].astype(I32)
    dst = edge_index[1].astype(I32)

    dst_s, src_s = lax.sort_key_val(dst, src)
    bounds = (jnp.arange(NWIN + 1, dtype=I32) * WIN)
    offs = jnp.searchsorted(dst_s, bounds).astype(I32)
    offs = jnp.concatenate(
        [offs, jnp.full((512 - NWIN - 1,), EE, I32)])
    src_p = jnp.concatenate([src_s, jnp.zeros((EPAD - EE,), I32)])
    dst_p = jnp.concatenate([dst_s, jnp.full((EPAD - EE,), NN, I32)])

    x1 = jnp.concatenate([pos, norm], axis=1)
    x1 = jnp.pad(x1, ((0, NPAD - NN), (0, 2)))
    w1a = jnp.pad(_augment_wide(W1, u1), ((0, 2), (0, 0)))

    g1, xu1 = _tc_matmul(x1, w1a, XU_WIDE)
    s1, cnt = _edge_stage(g1, xu1, src_p, dst_p, offs, c1, narrow=False)

    g2, xu2 = _tc_fused(s1, cnt, b1.reshape(1, -1), _augment_wide(W2, u2),
                        XU_WIDE)
    s2, _ = _edge_stage(g2, xu2, src_p, dst_p, offs, c2, narrow=False)

    g3, xu3 = _tc_fused(s2, cnt, b2.reshape(1, -1), _augment_wide(W3, u3),
                        XU_WIDE)
    s3, _ = _edge_stage(g3, xu3, src_p, dst_p, offs, c3, narrow=False)

    g4, xu4 = _tc_fused(s3, cnt, b3.reshape(1, -1), _augment_narrow(W4, u4),
                        XU_NARR)
    s4, _ = _edge_stage(g4, xu4, src_p, dst_p, offs, c4, narrow=True)

    b4p = jnp.zeros((1, 16), F32).at[0, :3].set(b4)
    out = _tc_final(s4, cnt, b4p)
    return out[:NN, :3]
